# baseline (speedup 1.0000x reference)
import functools

import jax
import jax.numpy as jnp
from jax import lax
from jax.experimental import pallas as pl
from jax.experimental.pallas import tpu as pltpu
from jax.experimental.pallas import tpu_sc as plsc

N = 10000
E = 320000
H = 128
T_ALL = 4
NB = 10
BLK = N // NB
CH = 128
NW = 32
NTILE = 16
NPAD = 10240
ROWS_PER_TILE = NPAD // NTILE
DEG_PER_TILE = NPAD // NTILE
NCHUNK = E // CH
ROWS_W = 80
LAST_W_ROWS = NCHUNK - 31 * ROWS_W


def _mesh():
    return plsc.VectorSubcoreMesh(core_axis_name="c", subcore_axis_name="s")


HROWS = ROWS_W // 2


def _build_aggregate3():
    @functools.partial(
        pl.kernel,
        mesh=_mesh(),
        out_type=jax.ShapeDtypeStruct((3, 2, NPAD, H), jnp.float32),
        scratch_types=[
            pltpu.VMEM((HROWS, CH), jnp.int32),
            pltpu.VMEM((HROWS, CH), jnp.int32),
            pltpu.VMEM((CH, H), jnp.float32),
            pltpu.VMEM((CH, H), jnp.float32),
            pltpu.VMEM_SHARED((NPAD, H), jnp.float32),
            pltpu.SemaphoreType.DMA,
            pltpu.SemaphoreType.DMA,
            pltpu.SemaphoreType.DMA,
            pltpu.SemaphoreType.DMA,
        ],
    )
    def agg(xs_hbm, ei_hbm, out_hbm,
            srcg, dstg, rows_a, rows_b, acc_sh,
            sem_a, sem_b, sem_sa, sem_sb):
        c = lax.axis_index("c")
        s = lax.axis_index("s")
        w = s * 2 + c
        row0 = w * ROWS_W
        nrows = jnp.where(w == NW - 1, LAST_W_ROWS, ROWS_W)
        base = s * ROWS_PER_TILE

        def npairs_of(half):
            return jnp.clip(nrows - half * HROWS, 0, HROWS) // 2

        def prep_half(t, half):
            hrow0 = row0 + half * HROWS
            pltpu.sync_copy(ei_hbm.at[t, 0, pl.ds(hrow0, HROWS)], srcg)
            pltpu.sync_copy(ei_hbm.at[t, 1, pl.ds(hrow0, HROWS)], dstg)
            if t > 0:
                toff = jnp.full((16,), t * N, jnp.int32)

                def offs(r, carry):
                    for jj in range(8):
                        sl = pl.ds(jj * 16, 16)
                        srcg[r, sl] = srcg[r, sl] + toff
                    return carry

                lax.fori_loop(0, HROWS, offs, 0)

            @pl.when(npairs_of(half) > 0)
            def _():
                pltpu.async_copy(xs_hbm.at[srcg.at[0]], rows_a, sem_a)

        def run_half(half):
            npairs = npairs_of(half)

            @pl.when(npairs > 0)
            def _():
                pltpu.async_copy(xs_hbm.at[srcg.at[1]], rows_b, sem_b)

            def chunk2(i, carry):
                j0 = 2 * i
                pltpu.make_async_copy(
                    xs_hbm.at[srcg.at[j0]], rows_a, sem_a).wait()
                pltpu.async_copy(rows_a, acc_sh.at[dstg.at[j0]], sem_sa,
                                 add=True)
                pltpu.make_async_copy(
                    xs_hbm.at[srcg.at[j0 + 1]], rows_b, sem_b).wait()
                pltpu.async_copy(rows_b, acc_sh.at[dstg.at[j0 + 1]], sem_sb,
                                 add=True)
                pltpu.make_async_copy(rows_a, acc_sh.at[dstg.at[j0]],
                                      sem_sa).wait()

                @pl.when(i < npairs - 1)
                def _():
                    pltpu.async_copy(
                        xs_hbm.at[srcg.at[j0 + 2]], rows_a, sem_a)

                pltpu.make_async_copy(rows_b, acc_sh.at[dstg.at[j0 + 1]],
                                      sem_sb).wait()

                @pl.when(i < npairs - 1)
                def _():
                    pltpu.async_copy(
                        xs_hbm.at[srcg.at[j0 + 3]], rows_b, sem_b)

                return carry

            lax.fori_loop(0, npairs, chunk2, 0)

        def zfill(i, carry):
            for jj in range(8):
                rows_b[i, pl.ds(jj * 16, 16)] = jnp.zeros((16,), jnp.float32)
            return carry

        def zero_acc():
            lax.fori_loop(0, CH, zfill, 0)
            for r in range(ROWS_PER_TILE // CH):
                pltpu.sync_copy(rows_b, acc_sh.at[pl.ds(base + r * CH, CH)])

        prep_half(0, 0)
        zero_acc()
        plsc.subcore_barrier()

        for t in range(3):
            run_half(0)
            prep_half(t, 1)
            run_half(1)
            plsc.subcore_barrier()
            if t < 2:
                prep_half(t + 1, 0)
            pltpu.sync_copy(acc_sh.at[pl.ds(base, ROWS_PER_TILE)],
                            out_hbm.at[t, c, pl.ds(base, ROWS_PER_TILE)])
            if t < 2:
                zero_acc()
            plsc.subcore_barrier()

    return agg


_AGG_CACHE = []


def _sc_aggregate3(xs_flat, eib):
    if not _AGG_CACHE:
        _AGG_CACHE.append(_build_aggregate3())
    return _AGG_CACHE[0](xs_flat, eib)


def _sc_degrees(eib):
    @functools.partial(
        pl.kernel,
        mesh=_mesh(),
        out_type=[jax.ShapeDtypeStruct((2 * NPAD,), jnp.float32)
                  for _ in range(3)],
        scratch_types=[
            pltpu.VMEM((ROWS_W, CH), jnp.int32),
            pltpu.VMEM((CH,), jnp.float32),
            pltpu.VMEM((DEG_PER_TILE,), jnp.float32),
            pltpu.VMEM_SHARED((NPAD,), jnp.float32),
            pltpu.VMEM_SHARED((NPAD,), jnp.float32),
            pltpu.VMEM_SHARED((NPAD,), jnp.float32),
            pltpu.SemaphoreType.DMA,
        ],
    )
    def deg(ei_hbm, o0, o1, o2, dstg, ones_v, zer_v, d0, d1, d2, sem_d):
        c = lax.axis_index("c")
        s = lax.axis_index("s")
        w = s * 2 + c
        row0 = w * ROWS_W
        nquads = jnp.where(w == NW - 1, LAST_W_ROWS // 4, ROWS_W // 4)
        degs = [d0, d1, d2]
        outs = [o0, o1, o2]

        for jj in range(8):
            ones_v[pl.ds(jj * 16, 16)] = jnp.ones((16,), jnp.float32)

        def zfill(i, carry):
            zer_v[pl.ds(i * 16, 16)] = jnp.zeros((16,), jnp.float32)
            return carry

        lax.fori_loop(0, DEG_PER_TILE // 16, zfill, 0)

        base = s * DEG_PER_TILE
        for t in range(3):
            pltpu.sync_copy(zer_v, degs[t].at[pl.ds(base, DEG_PER_TILE)])
        plsc.subcore_barrier()

        for t in range(3):
            pltpu.sync_copy(ei_hbm.at[t, 1, pl.ds(row0, ROWS_W)], dstg)

            def grp(i, carry):
                for jj in range(4):
                    pltpu.async_copy(ones_v, degs[t].at[dstg.at[i * 4 + jj]],
                                     sem_d, add=True)
                for jj in range(4):
                    pltpu.make_async_copy(
                        ones_v, degs[t].at[dstg.at[i * 4 + jj]],
                        sem_d).wait()
                return carry

            lax.fori_loop(0, nquads, grp, 0)

        plsc.subcore_barrier()
        for t in range(3):
            pltpu.sync_copy(degs[t].at[pl.ds(base, DEG_PER_TILE)],
                            outs[t].at[pl.ds(c * NPAD + base, DEG_PER_TILE)])

    return deg(eib)


def _evolve_body(wa_ref, wb_ref, wiha_ref, wihb_ref,
                 bia_ref, bib_ref, bha_ref, bhb_ref, out_ref):
    l = pl.program_id(0)
    W = jnp.where(l == 0, wa_ref[...], wb_ref[...])
    wih = jnp.where(l == 0, wiha_ref[...], wihb_ref[...])
    b = jnp.where(l == 0, bia_ref[...] + bha_ref[...],
                  bib_ref[...] + bhb_ref[...])
    c = jnp.zeros((H, H), jnp.float32)
    for s in range(3):
        g = lax.dot_general(W, wih, (((1,), (1,)), ((), ())),
                            preferred_element_type=jnp.float32) + b
        i_g = jax.nn.sigmoid(g[:, 0 * H:1 * H])
        f_g = jax.nn.sigmoid(g[:, 1 * H:2 * H])
        g_g = jnp.tanh(g[:, 2 * H:3 * H])
        o_g = jax.nn.sigmoid(g[:, 3 * H:4 * H])
        c = f_g * c + i_g * g_g
        W = o_g * jnp.tanh(c)
        out_ref[0, s] = W
    del out_ref


def _evolve_weights(wa, wb, wiha, wihb, bia, bib, bha, bhb):
    full = [
        pl.BlockSpec((H, H), lambda l: (0, 0)),
        pl.BlockSpec((H, H), lambda l: (0, 0)),
        pl.BlockSpec((4 * H, H), lambda l: (0, 0)),
        pl.BlockSpec((4 * H, H), lambda l: (0, 0)),
        pl.BlockSpec((1, 4 * H), lambda l: (0, 0)),
        pl.BlockSpec((1, 4 * H), lambda l: (0, 0)),
        pl.BlockSpec((1, 4 * H), lambda l: (0, 0)),
        pl.BlockSpec((1, 4 * H), lambda l: (0, 0)),
    ]
    return pl.pallas_call(
        _evolve_body,
        grid=(2,),
        in_specs=full,
        out_specs=pl.BlockSpec((1, 3, H, H), lambda l: (l, 0, 0, 0)),
        out_shape=jax.ShapeDtypeStruct((2, 3, H, H), jnp.float32),
    )(wa, wb, wiha, wihb, bia, bib, bha, bhb)


def _xs_body(x_ref, w_ref, d0_ref, d1_ref, d2_ref, o_ref, d_ref):
    t = pl.program_id(0) // NB
    dp = jnp.where(t == 0, d0_ref[...],
                   jnp.where(t == 1, d1_ref[...], d2_ref[...]))
    d = lax.rsqrt(dp[0] + dp[1] + 1.0)
    d_ref[...] = d
    o_ref[...] = jnp.dot(x_ref[...], w_ref[0, 0],
                         preferred_element_type=jnp.float32) * d


def _tc_xs(x4, W0s, d0, d1, d2):
    return pl.pallas_call(
        _xs_body,
        grid=(3 * NB,),
        in_specs=[
            pl.BlockSpec((BLK, H), lambda i: (i, 0)),
            pl.BlockSpec((1, 1, H, H), lambda i: (0, i // NB, 0, 0)),
            pl.BlockSpec((2, BLK, 1), lambda i: (0, i % NB, 0)),
            pl.BlockSpec((2, BLK, 1), lambda i: (0, i % NB, 0)),
            pl.BlockSpec((2, BLK, 1), lambda i: (0, i % NB, 0)),
        ],
        out_specs=[
            pl.BlockSpec((BLK, H), lambda i: (i, 0)),
            pl.BlockSpec((BLK, 1), lambda i: (i, 0)),
        ],
        out_shape=[
            jax.ShapeDtypeStruct((3 * N, H), jnp.float32),
            jax.ShapeDtypeStruct((3 * N, 1), jnp.float32),
        ],
    )(x4, W0s, d0, d1, d2)


def _comb_stats_body(p_ref, xs_ref, d_ref, b_ref, z_ref, s_ref, q_ref):
    z = (p_ref[0, 0] + p_ref[0, 1] + xs_ref[...]) * d_ref[...] + b_ref[...]
    z_ref[...] = z
    s_ref[0] = jnp.sum(z, 0, keepdims=True)
    q_ref[0] = jnp.sum(z * z, 0, keepdims=True)


def _tc_comb_stats(parts, xs_flat, dinv_flat, brow):
    return pl.pallas_call(
        _comb_stats_body,
        grid=(3 * NB,),
        in_specs=[
            pl.BlockSpec((1, 2, BLK, H), lambda i: (i // NB, 0, i % NB, 0)),
            pl.BlockSpec((BLK, H), lambda i: (i, 0)),
            pl.BlockSpec((BLK, 1), lambda i: (i, 0)),
            pl.BlockSpec((1, H), lambda i: (0, 0)),
        ],
        out_specs=[
            pl.BlockSpec((BLK, H), lambda i: (i, 0)),
            pl.BlockSpec((1, 1, H), lambda i: (i, 0, 0)),
            pl.BlockSpec((1, 1, H), lambda i: (i, 0, 0)),
        ],
        out_shape=[
            jax.ShapeDtypeStruct((3 * N, H), jnp.float32),
            jax.ShapeDtypeStruct((3 * NB, 1, H), jnp.float32),
            jax.ShapeDtypeStruct((3 * NB, 1, H), jnp.float32),
        ],
    )(parts, xs_flat, dinv_flat, brow)


def _bn_mm_body(z_ref, s_ref, q_ref, g_ref, be_ref, w_ref, d_ref, o_ref):
    t = pl.program_id(0) // NB
    rows = lax.broadcasted_iota(jnp.int32, (3 * NB, 1, 1), 0)
    mask = (rows >= t * NB) & (rows < (t + 1) * NB)
    mean = jnp.sum(jnp.where(mask, s_ref[...], 0.0), 0) * (1.0 / N)
    var = jnp.sum(jnp.where(mask, q_ref[...], 0.0), 0) * (1.0 / N) - mean * mean
    zn = (z_ref[...] - mean) * lax.rsqrt(var + 1e-5) * g_ref[...] + be_ref[...]
    zr = jnp.maximum(zn, 0.0)
    o_ref[...] = jnp.dot(zr, w_ref[0, 0],
                         preferred_element_type=jnp.float32) * d_ref[...]


def _tc_bn_mm(z_flat, psum, psumsq, grow, berow, W1s, dinv_flat):
    return pl.pallas_call(
        _bn_mm_body,
        grid=(3 * NB,),
        in_specs=[
            pl.BlockSpec((BLK, H), lambda i: (i, 0)),
            pl.BlockSpec((3 * NB, 1, H), lambda i: (0, 0, 0)),
            pl.BlockSpec((3 * NB, 1, H), lambda i: (0, 0, 0)),
            pl.BlockSpec((1, H), lambda i: (0, 0)),
            pl.BlockSpec((1, H), lambda i: (0, 0)),
            pl.BlockSpec((1, 1, H, H), lambda i: (1, i // NB, 0, 0)),
            pl.BlockSpec((BLK, 1), lambda i: (i, 0)),
        ],
        out_specs=pl.BlockSpec((BLK, H), lambda i: (i, 0)),
        out_shape=jax.ShapeDtypeStruct((3 * N, H), jnp.float32),
    )(z_flat, psum, psumsq, grow, berow, W1s, dinv_flat)


def _comb_head_body(p0_ref, p1_ref, p2_ref, x0_ref, x1_ref, x2_ref,
                    e0_ref, e1_ref, e2_ref, b_ref,
                    wi_ref, wh_ref, bi_ref, bh_ref,
                    o0_ref, o1_ref, o2_ref, oh_ref):
    b = b_ref[...]
    zs = []
    for p_ref, x_ref, e_ref, o_ref in (
            (p0_ref, x0_ref, e0_ref, o0_ref),
            (p1_ref, x1_ref, e1_ref, o1_ref),
            (p2_ref, x2_ref, e2_ref, o2_ref)):
        z = (p_ref[0, 0] + p_ref[0, 1] + x_ref[...]) * e_ref[...] + b
        o_ref[...] = z
        zs.append(z)

    wi = wi_ref[...]
    wh = wh_ref[...]
    fb = bi_ref[...] + bh_ref[...]

    def dotT(a, w):
        return lax.dot_general(a, w, (((1,), (1,)), ((), ())),
                               preferred_element_type=jnp.float32)

    def gates(g, cprev):
        i_g = jax.nn.sigmoid(g[:, 0 * H:1 * H])
        f_g = jax.nn.sigmoid(g[:, 1 * H:2 * H])
        g_g = jnp.tanh(g[:, 2 * H:3 * H])
        o_g = jax.nn.sigmoid(g[:, 3 * H:4 * H])
        c = f_g * cprev + i_g * g_g
        return c, o_g * jnp.tanh(c)

    g = dotT(zs[0], wi) + fb
    c, h = gates(g, jnp.zeros((BLK, H), jnp.float32))
    for z in zs[1:]:
        g = dotT(z, wi) + dotT(h, wh) + fb
        c, h = gates(g, c)
    g = dotT(h, wi) + fb
    _, h = gates(g, c)
    oh_ref[...] = h


def _tc_comb_head(parts, xs_flat, dinv_flat, brow, wih, whh, bir, bhr):
    parts_specs = [
        pl.BlockSpec((1, 2, BLK, H), lambda i, t=t: (t, 0, i, 0))
        for t in range(3)]
    flat_specs = [
        pl.BlockSpec((BLK, H), lambda i, t=t: (i + t * NB, 0))
        for t in range(3)]
    dinv_specs = [
        pl.BlockSpec((BLK, 1), lambda i, t=t: (i + t * NB, 0))
        for t in range(3)]
    return pl.pallas_call(
        _comb_head_body,
        grid=(NB,),
        in_specs=parts_specs + flat_specs + dinv_specs + [
            pl.BlockSpec((1, H), lambda i: (0, 0)),
            pl.BlockSpec((4 * H, H), lambda i: (0, 0)),
            pl.BlockSpec((4 * H, H), lambda i: (0, 0)),
            pl.BlockSpec((1, 4 * H), lambda i: (0, 0)),
            pl.BlockSpec((1, 4 * H), lambda i: (0, 0)),
        ],
        out_specs=[pl.BlockSpec((BLK, H), lambda i: (i, 0))] * 4,
        out_shape=[jax.ShapeDtypeStruct((N, H), jnp.float32)] * 4,
    )(parts, parts, parts, xs_flat, xs_flat, xs_flat,
      dinv_flat, dinv_flat, dinv_flat, brow, wih, whh, bir, bhr)


def kernel(x_seq, edge_index_seq, conv_w0, conv_b0, conv_w1, conv_b1,
           lstm0_wih, lstm0_whh, lstm0_bih, lstm0_bhh,
           lstm1_wih, lstm1_whh, lstm1_bih, lstm1_bhh,
           flstm_wih, flstm_whh, flstm_bih, flstm_bhh,
           bn_gamma, bn_beta):
    Wls = _evolve_weights(conv_w0, conv_w1, lstm0_wih, lstm1_wih,
                          lstm0_bih.reshape(1, 4 * H),
                          lstm1_bih.reshape(1, 4 * H),
                          lstm0_bhh.reshape(1, 4 * H),
                          lstm1_bhh.reshape(1, 4 * H))
    W0s = Wls
    W1s = Wls

    eib = edge_index_seq.reshape(T_ALL, 2, NCHUNK, CH)

    degs = _sc_degrees(eib)
    d0, d1, d2 = (d.reshape(2, NPAD, 1) for d in degs)

    b0r = conv_b0.reshape(1, H)
    b1r = conv_b1.reshape(1, H)
    grow = bn_gamma.reshape(1, H)
    berow = bn_beta.reshape(1, H)

    x4 = x_seq.reshape(4 * N, H)
    xs0, dinv = _tc_xs(x4, W0s, d0, d1, d2)
    parts0 = _sc_aggregate3(xs0, eib)
    z1pre, psum, psumsq = _tc_comb_stats(parts0, xs0, dinv, b0r)
    xs1 = _tc_bn_mm(z1pre, psum, psumsq, grow, berow, W1s, dinv)
    parts1 = _sc_aggregate3(xs1, eib)
    z0, z1, z2, head = _tc_comb_head(
        parts1, xs1, dinv, b1r, flstm_wih, flstm_whh,
        flstm_bih.reshape(1, 4 * H), flstm_bhh.reshape(1, 4 * H))
    return (z0, z1, z2, head)


# speedup vs baseline: 1.0759x; 1.0759x over previous
import functools

import jax
import jax.numpy as jnp
from jax import lax
from jax.experimental import pallas as pl
from jax.experimental.pallas import tpu as pltpu
from jax.experimental.pallas import tpu_sc as plsc

N = 10000
E = 320000
H = 128
T_ALL = 4
NB = 10
BLK = N // NB
CH = 128
NW = 32
NTILE = 16
NPAD = 10240
ROWS_PER_TILE = NPAD // NTILE
DEG_PER_TILE = NPAD // NTILE
NCHUNK = E // CH
ROWS_W = 80
LAST_W_ROWS = NCHUNK - 31 * ROWS_W


def _mesh():
    return plsc.VectorSubcoreMesh(core_axis_name="c", subcore_axis_name="s")


HROWS = ROWS_W // 2


def _build_aggregate3():
    @functools.partial(
        pl.kernel,
        mesh=_mesh(),
        out_type=jax.ShapeDtypeStruct((3, 2, NPAD, H), jnp.float32),
        scratch_types=[
            pltpu.VMEM((HROWS, CH), jnp.int32),
            pltpu.VMEM((HROWS, CH), jnp.int32),
            pltpu.VMEM((CH, H), jnp.float32),
            pltpu.VMEM((CH, H), jnp.float32),
            pltpu.VMEM_SHARED((NPAD, H), jnp.float32),
            pltpu.SemaphoreType.DMA,
            pltpu.SemaphoreType.DMA,
        ],
    )
    def agg(xs_hbm, ei_hbm, out_hbm,
            srcg, dstg, rows_a, rows_b, acc_sh, sem_a, sem_b):
        c = lax.axis_index("c")
        s = lax.axis_index("s")
        w = s * 2 + c
        row0 = w * ROWS_W
        nrows = jnp.where(w == NW - 1, LAST_W_ROWS, ROWS_W)
        base = s * ROWS_PER_TILE

        def npairs_of(half):
            return jnp.clip(nrows - half * HROWS, 0, HROWS) // 2

        def prep_half(t, half):
            hrow0 = row0 + half * HROWS
            pltpu.sync_copy(ei_hbm.at[t, 0, pl.ds(hrow0, HROWS)], srcg)
            pltpu.sync_copy(ei_hbm.at[t, 1, pl.ds(hrow0, HROWS)], dstg)
            if t > 0:
                toff = jnp.full((16,), t * N, jnp.int32)

                def offs(r, carry):
                    for jj in range(8):
                        sl = pl.ds(jj * 16, 16)
                        srcg[r, sl] = srcg[r, sl] + toff
                    return carry

                lax.fori_loop(0, HROWS, offs, 0)

            @pl.when(npairs_of(half) > 0)
            def _():
                pltpu.async_copy(xs_hbm.at[srcg.at[0]], rows_a, sem_a)

        def run_half(half):
            npairs = npairs_of(half)

            def chunk2(i, carry):
                j0 = 2 * i
                pltpu.make_async_copy(
                    xs_hbm.at[srcg.at[j0]], rows_a, sem_a).wait()
                pltpu.async_copy(
                    xs_hbm.at[srcg.at[j0 + 1]], rows_b, sem_b)
                pltpu.sync_copy(rows_a, acc_sh.at[dstg.at[j0]], add=True)
                pltpu.make_async_copy(
                    xs_hbm.at[srcg.at[j0 + 1]], rows_b, sem_b).wait()

                @pl.when(i < npairs - 1)
                def _():
                    pltpu.async_copy(
                        xs_hbm.at[srcg.at[j0 + 2]], rows_a, sem_a)

                pltpu.sync_copy(rows_b, acc_sh.at[dstg.at[j0 + 1]],
                                add=True)
                return carry

            lax.fori_loop(0, npairs, chunk2, 0)

        def zfill(i, carry):
            for jj in range(8):
                rows_b[i, pl.ds(jj * 16, 16)] = jnp.zeros((16,), jnp.float32)
            return carry

        def zero_acc():
            lax.fori_loop(0, CH, zfill, 0)
            for r in range(ROWS_PER_TILE // CH):
                pltpu.sync_copy(rows_b, acc_sh.at[pl.ds(base + r * CH, CH)])

        prep_half(0, 0)
        zero_acc()
        plsc.subcore_barrier()

        for t in range(3):
            run_half(0)
            prep_half(t, 1)
            run_half(1)
            plsc.subcore_barrier()
            if t < 2:
                prep_half(t + 1, 0)
            pltpu.sync_copy(acc_sh.at[pl.ds(base, ROWS_PER_TILE)],
                            out_hbm.at[t, c, pl.ds(base, ROWS_PER_TILE)])
            if t < 2:
                zero_acc()
            plsc.subcore_barrier()

    return agg


_AGG_CACHE = []


def _sc_aggregate3(xs_flat, eib):
    if not _AGG_CACHE:
        _AGG_CACHE.append(_build_aggregate3())
    return _AGG_CACHE[0](xs_flat, eib)


def _sc_degrees(eib):
    @functools.partial(
        pl.kernel,
        mesh=_mesh(),
        out_type=[jax.ShapeDtypeStruct((2 * NPAD,), jnp.float32)
                  for _ in range(3)],
        scratch_types=[
            pltpu.VMEM((ROWS_W, CH), jnp.int32),
            pltpu.VMEM((CH,), jnp.float32),
            pltpu.VMEM((DEG_PER_TILE,), jnp.float32),
            pltpu.VMEM_SHARED((NPAD,), jnp.float32),
            pltpu.VMEM_SHARED((NPAD,), jnp.float32),
            pltpu.VMEM_SHARED((NPAD,), jnp.float32),
            pltpu.SemaphoreType.DMA,
        ],
    )
    def deg(ei_hbm, o0, o1, o2, dstg, ones_v, zer_v, d0, d1, d2, sem_d):
        c = lax.axis_index("c")
        s = lax.axis_index("s")
        w = s * 2 + c
        row0 = w * ROWS_W
        nquads = jnp.where(w == NW - 1, LAST_W_ROWS // 4, ROWS_W // 4)
        degs = [d0, d1, d2]
        outs = [o0, o1, o2]

        for jj in range(8):
            ones_v[pl.ds(jj * 16, 16)] = jnp.ones((16,), jnp.float32)

        def zfill(i, carry):
            zer_v[pl.ds(i * 16, 16)] = jnp.zeros((16,), jnp.float32)
            return carry

        lax.fori_loop(0, DEG_PER_TILE // 16, zfill, 0)

        base = s * DEG_PER_TILE
        for t in range(3):
            pltpu.sync_copy(zer_v, degs[t].at[pl.ds(base, DEG_PER_TILE)])
        plsc.subcore_barrier()

        for t in range(3):
            pltpu.sync_copy(ei_hbm.at[t, 1, pl.ds(row0, ROWS_W)], dstg)

            def grp(i, carry):
                for jj in range(4):
                    pltpu.async_copy(ones_v, degs[t].at[dstg.at[i * 4 + jj]],
                                     sem_d, add=True)
                for jj in range(4):
                    pltpu.make_async_copy(
                        ones_v, degs[t].at[dstg.at[i * 4 + jj]],
                        sem_d).wait()
                return carry

            lax.fori_loop(0, nquads, grp, 0)

        plsc.subcore_barrier()
        for t in range(3):
            pltpu.sync_copy(degs[t].at[pl.ds(base, DEG_PER_TILE)],
                            outs[t].at[pl.ds(c * NPAD + base, DEG_PER_TILE)])

    return deg(eib)


def _evolve_body(wa_ref, wb_ref, wiha_ref, wihb_ref,
                 bia_ref, bib_ref, bha_ref, bhb_ref, out_ref):
    l = pl.program_id(0)
    W = jnp.where(l == 0, wa_ref[...], wb_ref[...])
    wih = jnp.where(l == 0, wiha_ref[...], wihb_ref[...])
    b = jnp.where(l == 0, bia_ref[...] + bha_ref[...],
                  bib_ref[...] + bhb_ref[...])
    c = jnp.zeros((H, H), jnp.float32)
    for s in range(3):
        g = lax.dot_general(W, wih, (((1,), (1,)), ((), ())),
                            preferred_element_type=jnp.float32) + b
        i_g = jax.nn.sigmoid(g[:, 0 * H:1 * H])
        f_g = jax.nn.sigmoid(g[:, 1 * H:2 * H])
        g_g = jnp.tanh(g[:, 2 * H:3 * H])
        o_g = jax.nn.sigmoid(g[:, 3 * H:4 * H])
        c = f_g * c + i_g * g_g
        W = o_g * jnp.tanh(c)
        out_ref[0, s] = W
    del out_ref


def _evolve_weights(wa, wb, wiha, wihb, bia, bib, bha, bhb):
    full = [
        pl.BlockSpec((H, H), lambda l: (0, 0)),
        pl.BlockSpec((H, H), lambda l: (0, 0)),
        pl.BlockSpec((4 * H, H), lambda l: (0, 0)),
        pl.BlockSpec((4 * H, H), lambda l: (0, 0)),
        pl.BlockSpec((1, 4 * H), lambda l: (0, 0)),
        pl.BlockSpec((1, 4 * H), lambda l: (0, 0)),
        pl.BlockSpec((1, 4 * H), lambda l: (0, 0)),
        pl.BlockSpec((1, 4 * H), lambda l: (0, 0)),
    ]
    return pl.pallas_call(
        _evolve_body,
        grid=(2,),
        in_specs=full,
        out_specs=pl.BlockSpec((1, 3, H, H), lambda l: (l, 0, 0, 0)),
        out_shape=jax.ShapeDtypeStruct((2, 3, H, H), jnp.float32),
    )(wa, wb, wiha, wihb, bia, bib, bha, bhb)


def _xs_body(x_ref, w_ref, d0_ref, d1_ref, d2_ref, o_ref, d_ref):
    t = pl.program_id(0) // NB
    dp = jnp.where(t == 0, d0_ref[...],
                   jnp.where(t == 1, d1_ref[...], d2_ref[...]))
    d = lax.rsqrt(dp[0] + dp[1] + 1.0)
    d_ref[...] = d
    o_ref[...] = jnp.dot(x_ref[...], w_ref[0, 0],
                         preferred_element_type=jnp.float32) * d


def _tc_xs(x4, W0s, d0, d1, d2):
    return pl.pallas_call(
        _xs_body,
        grid=(3 * NB,),
        in_specs=[
            pl.BlockSpec((BLK, H), lambda i: (i, 0)),
            pl.BlockSpec((1, 1, H, H), lambda i: (0, i // NB, 0, 0)),
            pl.BlockSpec((2, BLK, 1), lambda i: (0, i % NB, 0)),
            pl.BlockSpec((2, BLK, 1), lambda i: (0, i % NB, 0)),
            pl.BlockSpec((2, BLK, 1), lambda i: (0, i % NB, 0)),
        ],
        out_specs=[
            pl.BlockSpec((BLK, H), lambda i: (i, 0)),
            pl.BlockSpec((BLK, 1), lambda i: (i, 0)),
        ],
        out_shape=[
            jax.ShapeDtypeStruct((3 * N, H), jnp.float32),
            jax.ShapeDtypeStruct((3 * N, 1), jnp.float32),
        ],
    )(x4, W0s, d0, d1, d2)


def _comb_stats_body(p_ref, xs_ref, d_ref, b_ref, z_ref, s_ref, q_ref):
    z = (p_ref[0, 0] + p_ref[0, 1] + xs_ref[...]) * d_ref[...] + b_ref[...]
    z_ref[...] = z
    s_ref[0] = jnp.sum(z, 0, keepdims=True)
    q_ref[0] = jnp.sum(z * z, 0, keepdims=True)


def _tc_comb_stats(parts, xs_flat, dinv_flat, brow):
    return pl.pallas_call(
        _comb_stats_body,
        grid=(3 * NB,),
        in_specs=[
            pl.BlockSpec((1, 2, BLK, H), lambda i: (i // NB, 0, i % NB, 0)),
            pl.BlockSpec((BLK, H), lambda i: (i, 0)),
            pl.BlockSpec((BLK, 1), lambda i: (i, 0)),
            pl.BlockSpec((1, H), lambda i: (0, 0)),
        ],
        out_specs=[
            pl.BlockSpec((BLK, H), lambda i: (i, 0)),
            pl.BlockSpec((1, 1, H), lambda i: (i, 0, 0)),
            pl.BlockSpec((1, 1, H), lambda i: (i, 0, 0)),
        ],
        out_shape=[
            jax.ShapeDtypeStruct((3 * N, H), jnp.float32),
            jax.ShapeDtypeStruct((3 * NB, 1, H), jnp.float32),
            jax.ShapeDtypeStruct((3 * NB, 1, H), jnp.float32),
        ],
    )(parts, xs_flat, dinv_flat, brow)


def _bn_mm_body(z_ref, s_ref, q_ref, g_ref, be_ref, w_ref, d_ref, o_ref):
    t = pl.program_id(0) // NB
    rows = lax.broadcasted_iota(jnp.int32, (3 * NB, 1, 1), 0)
    mask = (rows >= t * NB) & (rows < (t + 1) * NB)
    mean = jnp.sum(jnp.where(mask, s_ref[...], 0.0), 0) * (1.0 / N)
    var = jnp.sum(jnp.where(mask, q_ref[...], 0.0), 0) * (1.0 / N) - mean * mean
    zn = (z_ref[...] - mean) * lax.rsqrt(var + 1e-5) * g_ref[...] + be_ref[...]
    zr = jnp.maximum(zn, 0.0)
    o_ref[...] = jnp.dot(zr, w_ref[0, 0],
                         preferred_element_type=jnp.float32) * d_ref[...]


def _tc_bn_mm(z_flat, psum, psumsq, grow, berow, W1s, dinv_flat):
    return pl.pallas_call(
        _bn_mm_body,
        grid=(3 * NB,),
        in_specs=[
            pl.BlockSpec((BLK, H), lambda i: (i, 0)),
            pl.BlockSpec((3 * NB, 1, H), lambda i: (0, 0, 0)),
            pl.BlockSpec((3 * NB, 1, H), lambda i: (0, 0, 0)),
            pl.BlockSpec((1, H), lambda i: (0, 0)),
            pl.BlockSpec((1, H), lambda i: (0, 0)),
            pl.BlockSpec((1, 1, H, H), lambda i: (1, i // NB, 0, 0)),
            pl.BlockSpec((BLK, 1), lambda i: (i, 0)),
        ],
        out_specs=pl.BlockSpec((BLK, H), lambda i: (i, 0)),
        out_shape=jax.ShapeDtypeStruct((3 * N, H), jnp.float32),
    )(z_flat, psum, psumsq, grow, berow, W1s, dinv_flat)


def _comb_head_body(p0_ref, p1_ref, p2_ref, x0_ref, x1_ref, x2_ref,
                    e0_ref, e1_ref, e2_ref, b_ref,
                    wi_ref, wh_ref, bi_ref, bh_ref,
                    o0_ref, o1_ref, o2_ref, oh_ref):
    b = b_ref[...]
    zs = []
    for p_ref, x_ref, e_ref, o_ref in (
            (p0_ref, x0_ref, e0_ref, o0_ref),
            (p1_ref, x1_ref, e1_ref, o1_ref),
            (p2_ref, x2_ref, e2_ref, o2_ref)):
        z = (p_ref[0, 0] + p_ref[0, 1] + x_ref[...]) * e_ref[...] + b
        o_ref[...] = z
        zs.append(z)

    wi = wi_ref[...]
    wh = wh_ref[...]
    fb = bi_ref[...] + bh_ref[...]

    def dotT(a, w):
        return lax.dot_general(a, w, (((1,), (1,)), ((), ())),
                               preferred_element_type=jnp.float32)

    def gates(g, cprev):
        i_g = jax.nn.sigmoid(g[:, 0 * H:1 * H])
        f_g = jax.nn.sigmoid(g[:, 1 * H:2 * H])
        g_g = jnp.tanh(g[:, 2 * H:3 * H])
        o_g = jax.nn.sigmoid(g[:, 3 * H:4 * H])
        c = f_g * cprev + i_g * g_g
        return c, o_g * jnp.tanh(c)

    g = dotT(zs[0], wi) + fb
    c, h = gates(g, jnp.zeros((BLK, H), jnp.float32))
    for z in zs[1:]:
        g = dotT(z, wi) + dotT(h, wh) + fb
        c, h = gates(g, c)
    g = dotT(h, wi) + fb
    _, h = gates(g, c)
    oh_ref[...] = h


def _tc_comb_head(parts, xs_flat, dinv_flat, brow, wih, whh, bir, bhr):
    parts_specs = [
        pl.BlockSpec((1, 2, BLK, H), lambda i, t=t: (t, 0, i, 0))
        for t in range(3)]
    flat_specs = [
        pl.BlockSpec((BLK, H), lambda i, t=t: (i + t * NB, 0))
        for t in range(3)]
    dinv_specs = [
        pl.BlockSpec((BLK, 1), lambda i, t=t: (i + t * NB, 0))
        for t in range(3)]
    return pl.pallas_call(
        _comb_head_body,
        grid=(NB,),
        in_specs=parts_specs + flat_specs + dinv_specs + [
            pl.BlockSpec((1, H), lambda i: (0, 0)),
            pl.BlockSpec((4 * H, H), lambda i: (0, 0)),
            pl.BlockSpec((4 * H, H), lambda i: (0, 0)),
            pl.BlockSpec((1, 4 * H), lambda i: (0, 0)),
            pl.BlockSpec((1, 4 * H), lambda i: (0, 0)),
        ],
        out_specs=[pl.BlockSpec((BLK, H), lambda i: (i, 0))] * 4,
        out_shape=[jax.ShapeDtypeStruct((N, H), jnp.float32)] * 4,
    )(parts, parts, parts, xs_flat, xs_flat, xs_flat,
      dinv_flat, dinv_flat, dinv_flat, brow, wih, whh, bir, bhr)


def kernel(x_seq, edge_index_seq, conv_w0, conv_b0, conv_w1, conv_b1,
           lstm0_wih, lstm0_whh, lstm0_bih, lstm0_bhh,
           lstm1_wih, lstm1_whh, lstm1_bih, lstm1_bhh,
           flstm_wih, flstm_whh, flstm_bih, flstm_bhh,
           bn_gamma, bn_beta):
    Wls = _evolve_weights(conv_w0, conv_w1, lstm0_wih, lstm1_wih,
                          lstm0_bih.reshape(1, 4 * H),
                          lstm1_bih.reshape(1, 4 * H),
                          lstm0_bhh.reshape(1, 4 * H),
                          lstm1_bhh.reshape(1, 4 * H))
    W0s = Wls
    W1s = Wls

    eib = edge_index_seq.reshape(T_ALL, 2, NCHUNK, CH)

    degs = _sc_degrees(eib)
    d0, d1, d2 = (d.reshape(2, NPAD, 1) for d in degs)

    b0r = conv_b0.reshape(1, H)
    b1r = conv_b1.reshape(1, H)
    grow = bn_gamma.reshape(1, H)
    berow = bn_beta.reshape(1, H)

    x4 = x_seq.reshape(4 * N, H)
    xs0, dinv = _tc_xs(x4, W0s, d0, d1, d2)
    parts0 = _sc_aggregate3(xs0, eib)
    z1pre, psum, psumsq = _tc_comb_stats(parts0, xs0, dinv, b0r)
    xs1 = _tc_bn_mm(z1pre, psum, psumsq, grow, berow, W1s, dinv)
    parts1 = _sc_aggregate3(xs1, eib)
    z0, z1, z2, head = _tc_comb_head(
        parts1, xs1, dinv, b1r, flstm_wih, flstm_whh,
        flstm_bih.reshape(1, 4 * H), flstm_bhh.reshape(1, 4 * H))
    return (z0, z1, z2, head)
